# Initial kernel scaffold; baseline (speedup 1.0000x reference)
import jax
import jax.numpy as jnp
from jax.experimental import pallas as pl


def kernel(indices, values, embedding):
    raise NotImplementedError("write your pallas kernel here")


# speedup vs baseline: 1.7876x; 1.7876x over previous
import functools

import jax
import jax.numpy as jnp
from jax import lax
from jax.experimental import pallas as pl
from jax.experimental.pallas import tpu as pltpu
from jax.experimental.pallas import tpu_sc as plsc

N_BUCKETS = 1000000
B, N, D = 4096, 200, 32
NC, NS = 2, 16
NW = NC * NS
BPW = B // NW
L = 16
NFULL = N // L
TAIL_OFF = N - L


def _sc_body(idx_hbm, val_hbm, emb_hbm, out_hbm,
             idx_v, val_v, w_v, rows_v, out_v, sem):
    wid = lax.axis_index("s") * NC + lax.axis_index("c")
    base = wid * BPW

    pltpu.sync_copy(idx_hbm.at[pl.ds(base, BPW)], idx_v)
    pltpu.sync_copy(val_hbm.at[pl.ds(base, BPW)], val_v)

    def do_row(r):
        def chunk(off):
            x = idx_v[r, pl.ds(off, L)]
            v = val_v[r, pl.ds(off, L)]
            bucket = lax.rem(x, N_BUCKETS)
            sign = (2 * (x & 1) - 1).astype(jnp.float32)
            idx_v[r, pl.ds(off, L)] = bucket
            w_v[r, pl.ds(off, L)] = sign * v

        for j in range(NFULL):
            chunk(j * L)
        chunk(TAIL_OFF)

        c0 = pltpu.async_copy(emb_hbm.at[idx_v.at[r, pl.ds(0, 128)]],
                              rows_v.at[pl.ds(0, 128)], sem)
        c1 = pltpu.async_copy(emb_hbm.at[idx_v.at[r, pl.ds(128, N - 128)]],
                              rows_v.at[pl.ds(128, N - 128)], sem)
        c0.wait()
        c1.wait()

        def acc_chunk(c, carry):
            a0, a1 = carry
            n0 = c * L
            wv = w_v[r, pl.ds(n0, L)]
            for k in range(L):
                w = wv[k]
                a0 = a0 + rows_v[n0 + k, pl.ds(0, L)] * w
                a1 = a1 + rows_v[n0 + k, pl.ds(L, L)] * w
            return a0, a1

        a0, a1 = lax.fori_loop(
            0, NFULL, acc_chunk,
            (jnp.zeros((L,), jnp.float32), jnp.zeros((L,), jnp.float32)))
        wv = w_v[r, pl.ds(TAIL_OFF, L)]
        for k in range(L - (N - NFULL * L), L):
            w = wv[k]
            a0 = a0 + rows_v[TAIL_OFF + k, pl.ds(0, L)] * w
            a1 = a1 + rows_v[TAIL_OFF + k, pl.ds(L, L)] * w
        out_v[r, pl.ds(0, L)] = a0
        out_v[r, pl.ds(L, L)] = a1

    lax.fori_loop(0, BPW, lambda r, _: (do_row(r), 0)[1], 0)

    pltpu.sync_copy(out_v, out_hbm.at[pl.ds(base, BPW)])


@jax.jit
def _fh_sc(indices, values, embedding):
    mesh = plsc.VectorSubcoreMesh(core_axis_name="c", subcore_axis_name="s",
                                  num_cores=NC, num_subcores=NS)
    return pl.kernel(
        _sc_body,
        out_type=jax.ShapeDtypeStruct((B, D), jnp.float32),
        mesh=mesh,
        compiler_params=pltpu.CompilerParams(use_tc_tiling_on_sc=False),
        scratch_types=[
            pltpu.VMEM((BPW, N), jnp.int32),
            pltpu.VMEM((BPW, N), jnp.float32),
            pltpu.VMEM((BPW, N), jnp.float32),
            pltpu.VMEM((N, D), jnp.float32),
            pltpu.VMEM((BPW, D), jnp.float32),
            pltpu.SemaphoreType.DMA,
        ],
    )(indices, values, embedding)


def kernel(indices, values, embedding):
    return _fh_sc(indices.astype(jnp.int32), values, embedding)


# speedup vs baseline: 2.1846x; 1.2221x over previous
import functools

import jax
import jax.numpy as jnp
from jax import lax
from jax.experimental import pallas as pl
from jax.experimental.pallas import tpu as pltpu
from jax.experimental.pallas import tpu_sc as plsc

N_BUCKETS = 1000000
B, N, D = 4096, 200, 32
NC, NS = 2, 16
NW = NC * NS
BPW = B // NW
L = 16
NFULL = N // L
TAIL = N - NFULL * L
TAIL_OFF = N - L
K = 8
G0 = 128
G1 = N - G0


def _sc_body(idx_hbm, val_hbm, emb_hbm, out_hbm, idx_v, w_v, rows_v, out_v, *sems):
    wid = lax.axis_index("s") * NC + lax.axis_index("c")
    base = wid * BPW

    pltpu.sync_copy(idx_hbm.at[pl.ds(base, BPW)], idx_v)
    pltpu.sync_copy(val_hbm.at[pl.ds(base, BPW)], w_v)

    def prep_row(r):
        def chunk(j, carry):
            off = j * L
            x = idx_v[r, pl.ds(off, L)]
            v = w_v[r, pl.ds(off, L)]
            idx_v[r, pl.ds(off, L)] = lax.rem(x, N_BUCKETS)
            w_v[r, pl.ds(off, L)] = (2 * (x & 1) - 1).astype(jnp.float32) * v
            return carry

        lax.fori_loop(0, NFULL, chunk, 0)
        x = idx_v[r, pl.ds(TAIL_OFF, L)]
        v = w_v[r, pl.ds(TAIL_OFF, L)]
        s = (2 * (x & 1) - 1).astype(jnp.float32)
        lane = lax.iota(jnp.int32, L)
        idx_v[r, pl.ds(TAIL_OFF, L)] = lax.rem(x, N_BUCKETS)
        w_v[r, pl.ds(TAIL_OFF, L)] = jnp.where(lane < (L - TAIL), v, s * v)

    def gather_parts(r, b):
        yield (emb_hbm.at[idx_v.at[r, pl.ds(0, G0)]],
               rows_v.at[b, pl.ds(0, G0)], sems[b])
        yield (emb_hbm.at[idx_v.at[r, pl.ds(G0, G1)]],
               rows_v.at[b, pl.ds(G0, G1)], sems[b])

    def issue(r, b):
        for src, dst, sem in gather_parts(r, b):
            pltpu.async_copy(src, dst, sem)

    def wait(r, b):
        for src, dst, sem in gather_parts(r, b):
            pltpu.make_async_copy(src, dst, sem).wait()

    def compute_row(r, b):
        def acc_chunk(c, carry):
            a0, a1 = carry
            n0 = c * L
            wv = w_v[r, pl.ds(n0, L)]
            for k in range(L):
                w = wv[k]
                a0 = a0 + rows_v[b, n0 + k, pl.ds(0, L)] * w
                a1 = a1 + rows_v[b, n0 + k, pl.ds(L, L)] * w
            return a0, a1

        a0, a1 = lax.fori_loop(
            0, NFULL, acc_chunk,
            (jnp.zeros((L,), jnp.float32), jnp.zeros((L,), jnp.float32)))
        wv = w_v[r, pl.ds(TAIL_OFF, L)]
        for k in range(L - TAIL, L):
            w = wv[k]
            a0 = a0 + rows_v[b, TAIL_OFF + k, pl.ds(0, L)] * w
            a1 = a1 + rows_v[b, TAIL_OFF + k, pl.ds(L, L)] * w
        out_v[r, pl.ds(0, L)] = a0
        out_v[r, pl.ds(L, L)] = a1

    lax.fori_loop(0, BPW, lambda r, c: (prep_row(r), c)[1], 0)

    for b in range(K):
        issue(b, b)

    def outer(g, carry):
        r0 = g * K
        for b in range(K):
            r = r0 + b
            wait(r, b)
            compute_row(r, b)
            nxt = r + K

            @pl.when(nxt < BPW)
            def _():
                issue(nxt, b)
        return carry

    lax.fori_loop(0, BPW // K, outer, 0)

    pltpu.sync_copy(out_v, out_hbm.at[pl.ds(base, BPW)])


@jax.jit
def _fh_sc(indices, values, embedding):
    mesh = plsc.VectorSubcoreMesh(core_axis_name="c", subcore_axis_name="s",
                                  num_cores=NC, num_subcores=NS)
    return pl.kernel(
        _sc_body,
        out_type=jax.ShapeDtypeStruct((B, D), jnp.float32),
        mesh=mesh,
        compiler_params=pltpu.CompilerParams(use_tc_tiling_on_sc=False),
        scratch_types=[
            pltpu.VMEM((BPW, N), jnp.int32),
            pltpu.VMEM((BPW, N), jnp.float32),
            pltpu.VMEM((K, N, D), jnp.float32),
            pltpu.VMEM((BPW, D), jnp.float32),
        ] + [pltpu.SemaphoreType.DMA] * K,
    )(indices, values, embedding)


def kernel(indices, values, embedding):
    return _fh_sc(indices.astype(jnp.int32), values, embedding)
